# baseline (speedup 1.0000x reference)
import functools

import jax
import jax.numpy as jnp
from jax import lax
from jax.experimental import pallas as pl
from jax.experimental.pallas import tpu as pltpu
from jax.experimental.pallas import tpu_sc as plsc

NUM_INST = 50000
N_TICKS = 3200000

NC = 2
NS = 16
L = 16
NW = NC * NS

C = 1568
NIDS = C * NW
BLK = 4096
ALIGNED = (N_TICKS // BLK) * BLK
TAIL = N_TICKS - ALIGNED
SEARCH_STEPS = 22

_i32 = jnp.int32
_f32 = jnp.float32


def _sc_body(ids_hbm, prc_hbm, cur_hbm, out_hbm,
             acc_hi0, acc_lo0,
             ids_buf0, ids_buf1, prc_buf0, prc_buf1,
             cur_buf, sig_buf, probe_buf, sem_probe, sem_blk0, sem_blk1,
             sem_cur):
    lane = jnp.arange(L, dtype=_i32)
    w = lax.axis_index("s") * NC + lax.axis_index("c")
    base = w * C

    neg_inf = jnp.full((L,), -jnp.inf, dtype=_f32)
    pos_inf = jnp.full((L,), jnp.inf, dtype=_f32)
    cur_cp = pltpu.async_copy(
        cur_hbm.at[pl.ds(pl.multiple_of(base, 8), C)], cur_buf, sem_cur)

    t_a = base
    t_b = base + C
    probe2 = ids_buf1.at[pl.ds(0, L)]
    lo_a = jnp.int32(0)
    hi_a = jnp.int32(N_TICKS)
    lo_b = jnp.int32(0)
    hi_b = jnp.int32(N_TICKS)
    fills = (314, 314, 314, 314, 312, 0)
    fill_base = 0
    for step in range(6):
        geometric = step < 5
        w_a = hi_a - lo_a
        w_b = hi_b - lo_b
        off_a = ((w_a * lane) >> 4) if geometric else lane
        off_b = ((w_b * lane) >> 4) if geometric else lane
        p_a = lo_a + off_a
        p_b = lo_b + off_b
        cpa = pltpu.async_copy(
            ids_hbm.at[jnp.minimum(p_a, N_TICKS - 1)], probe_buf, sem_probe)
        cpb = pltpu.async_copy(
            ids_hbm.at[jnp.minimum(p_b, N_TICKS - 1)], probe2, sem_blk1)
        if fills[step]:
            fb = fill_base

            def init_body(j, carry, fb=fb):
                o = pl.multiple_of((fb + j) * L, L)
                acc_hi0[pl.ds(o, L)] = neg_inf
                acc_lo0[pl.ds(o, L)] = pos_inf
                return carry

            lax.fori_loop(0, fills[step], init_body, 0)
            fill_base += fills[step]
        cpa.wait()
        cpb.wait()
        g_a = probe_buf[...]
        g_b = ids_buf1[pl.ds(0, L)]
        m_a = plsc.all_reduce_population_count((g_a < t_a) & (off_a < w_a))[0]
        m_b = plsc.all_reduce_population_count((g_b < t_b) & (off_b < w_b))[0]
        if geometric:
            nlo_a = jnp.where(m_a > 0, lo_a + ((w_a * (m_a - 1)) >> 4) + 1, lo_a)
            hi_a = jnp.where(m_a < L, lo_a + ((w_a * m_a) >> 4), hi_a)
            lo_a = nlo_a
            nlo_b = jnp.where(m_b > 0, lo_b + ((w_b * (m_b - 1)) >> 4) + 1, lo_b)
            hi_b = jnp.where(m_b < L, lo_b + ((w_b * m_b) >> 4), hi_b)
            lo_b = nlo_b
        else:
            lo_a = lo_a + m_a
            lo_b = lo_b + m_b
    t0 = lo_a
    t1 = lo_b

    start = (t0 // BLK) * BLK
    t1c = jnp.minimum(t1, ALIGNED)
    nblk = jnp.maximum((t1c - start + (BLK - 1)) // BLK, 0)

    UNROLL = 8

    bufs = ((ids_buf0, prc_buf0), (ids_buf1, prc_buf1))

    def issue(bi, slot, sem):
        off = jnp.minimum(start + bi * BLK, ALIGNED - BLK)
        off = pl.multiple_of(off, 8)
        pltpu.async_copy(ids_hbm.at[pl.ds(off, BLK)], bufs[slot][0], sem)
        pltpu.async_copy(prc_hbm.at[pl.ds(off, BLK)], bufs[slot][1], sem)

    def wait_blk(slot, sem):
        pltpu.make_async_copy(
            ids_hbm.at[pl.ds(0, BLK)], bufs[slot][0], sem).wait()
        pltpu.make_async_copy(
            prc_hbm.at[pl.ds(0, BLK)], bufs[slot][1], sem).wait()

    def flush(pid, prm, prn, cont):
        ploc = pid - base
        pvalid = (ploc >= 0) & (ploc < C)
        pslot = jnp.where(pvalid, ploc, 0) * L + lane
        fmask = jnp.logical_not(cont) & pvalid
        plsc.store_scatter(acc_hi0, [pslot], prm, mask=fmask)
        plsc.store_scatter(acc_lo0, [pslot], prn, mask=fmask)

    def step_vec(idv, pv, carry):
        pid, prm, prn = carry
        cont = idv == pid
        flush(pid, prm, prn, cont)
        rm = jnp.maximum(jnp.where(cont, prm, neg_inf), pv)
        rn = jnp.minimum(jnp.where(cont, prn, pos_inf), pv)
        return idv, rm, rn

    def process_block(slot, carry):
        def vec_body(i, c):
            for u in range(UNROLL):
                iu = i * UNROLL + u
                voff = pl.multiple_of(iu * L, L)
                idv = bufs[slot][0][pl.ds(voff, L)]
                pv = bufs[slot][1][pl.ds(voff, L)]
                c = step_vec(idv, pv, c)
            return c

        return lax.fori_loop(0, BLK // (L * UNROLL), vec_body, carry)

    def maybe_process(slot, cond, carry):
        return lax.cond(cond, lambda c: process_block(slot, c),
                        lambda c: c, carry)

    nb2 = (jnp.maximum(nblk, 1) + 1) >> 1
    issue(0, 0, sem_blk0)
    sentinel = jnp.full((L,), -1, dtype=_i32)
    carry0 = (sentinel, neg_inf, pos_inf)

    def pair_body(p2, carry):
        b0 = 2 * p2
        issue(b0 + 1, 1, sem_blk1)
        wait_blk(0, sem_blk0)
        carry = maybe_process(0, b0 < nblk, carry)

        @pl.when(p2 + 1 < nb2)
        def _prefetch():
            issue(b0 + 2, 0, sem_blk0)

        wait_blk(1, sem_blk1)
        carry = maybe_process(1, b0 + 1 < nblk, carry)
        return carry

    pid_f, prm_f, prn_f = lax.fori_loop(0, nb2, pair_body, carry0)
    flush(pid_f, prm_f, prn_f, jnp.zeros((L,), dtype=jnp.bool_))

    pltpu.async_copy(ids_hbm.at[pl.ds(ALIGNED, TAIL)],
                     ids_buf0.at[pl.ds(0, TAIL)], sem_blk0)
    pltpu.async_copy(prc_hbm.at[pl.ds(ALIGNED, TAIL)],
                     prc_buf0.at[pl.ds(0, TAIL)], sem_blk0)
    pltpu.make_async_copy(ids_hbm.at[pl.ds(0, TAIL)],
                          ids_buf0.at[pl.ds(0, TAIL)], sem_blk0).wait()
    pltpu.make_async_copy(prc_hbm.at[pl.ds(0, TAIL)],
                          prc_buf0.at[pl.ds(0, TAIL)], sem_blk0).wait()

    def tail_body(i, carry):
        for u in range(UNROLL):
            iu = i * UNROLL + u
            voff = pl.multiple_of(iu * L, L)
            idv = ids_buf0[pl.ds(voff, L)]
            pv = prc_buf0[pl.ds(voff, L)]
            loc = idv - base
            valid = (loc >= 0) & (loc < C)
            locc = jnp.where(valid, loc, 0)
            perm = (lane + iu) & (L - 1)
            slot_v = locc * L + perm
            h = plsc.load_gather(acc_hi0, [slot_v])
            lw = plsc.load_gather(acc_lo0, [slot_v])
            plsc.store_scatter(acc_hi0, [slot_v], jnp.maximum(h, pv), mask=valid)
            plsc.store_scatter(acc_lo0, [slot_v], jnp.minimum(lw, pv), mask=valid)
        return carry

    lax.fori_loop(0, TAIL // (L * UNROLL), tail_body, 0)

    one = jnp.float32(1.0)
    zero = jnp.float32(0.0)

    cur_cp.wait()

    def comb_body(j, carry):
        joff = j * L
        rowidx = (joff + lane) * L
        h = None
        lw = None
        for p in range(L):
            sk = rowidx + ((lane + p) & (L - 1))
            hp = plsc.load_gather(acc_hi0, [sk])
            lp = plsc.load_gather(acc_lo0, [sk])
            h = hp if h is None else jnp.maximum(h, hp)
            lw = lp if lw is None else jnp.minimum(lw, lp)
        empty = h == -jnp.inf
        h = jnp.where(empty, zero, h)
        lw = jnp.where(empty, zero, lw)
        cur = cur_buf[pl.ds(pl.multiple_of(joff, L), L)]
        sig = jnp.where(cur > h, one, jnp.where(cur < lw, -one, zero))
        sig_buf[pl.ds(pl.multiple_of(joff, L), L)] = sig
        return carry

    lax.fori_loop(0, C // L, comb_body, 0)
    pltpu.sync_copy(sig_buf, out_hbm.at[pl.ds(pl.multiple_of(base, 8), C)])


@jax.jit
def _run(inst_ids, tick_price, cur_price):
    mesh = plsc.VectorSubcoreMesh(core_axis_name="c", subcore_axis_name="s")
    kern = functools.partial(
        pl.kernel,
        mesh=mesh,
        compiler_params=pltpu.CompilerParams(needs_layout_passes=False),
        out_type=jax.ShapeDtypeStruct((NIDS,), _f32),
        scratch_types=[
            pltpu.VMEM((L * C,), _f32),
            pltpu.VMEM((L * C,), _f32),
            pltpu.VMEM((BLK,), _i32),
            pltpu.VMEM((BLK,), _i32),
            pltpu.VMEM((BLK,), _f32),
            pltpu.VMEM((BLK,), _f32),
            pltpu.VMEM((C,), _f32),
            pltpu.VMEM((C,), _f32),
            pltpu.VMEM((L,), _i32),
            pltpu.SemaphoreType.DMA,
            pltpu.SemaphoreType.DMA,
            pltpu.SemaphoreType.DMA,
            pltpu.SemaphoreType.DMA,
        ],
    )(_sc_body)
    cur_pad = jnp.concatenate(
        [cur_price, jnp.zeros((NIDS - NUM_INST,), dtype=_f32)])
    out = kern(inst_ids, tick_price, cur_pad)
    return out[:NUM_INST]


def kernel(timestamp, inst_ids, tick_price, cur_price):
    del timestamp
    return _run(inst_ids.astype(_i32), tick_price, cur_price)


# speedup vs baseline: 1.0136x; 1.0136x over previous
import functools

import jax
import jax.numpy as jnp
from jax import lax
from jax.experimental import pallas as pl
from jax.experimental.pallas import tpu as pltpu
from jax.experimental.pallas import tpu_sc as plsc

NUM_INST = 50000
N_TICKS = 3200000

NC = 2
NS = 16
L = 16
NW = NC * NS

C = 1568
NIDS = C * NW
BLK = 4096
ALIGNED = (N_TICKS // BLK) * BLK
TAIL = N_TICKS - ALIGNED
SEARCH_STEPS = 22

_i32 = jnp.int32
_f32 = jnp.float32


def _sc_body(ids_hbm, prc_hbm, cur_hbm, out_hbm,
             acc_hi0, acc_lo0,
             ids_buf0, ids_buf1, prc_buf0, prc_buf1,
             cur_buf, sig_buf, probe_buf, sem_probe, sem_blk0, sem_blk1,
             sem_cur):
    lane = jnp.arange(L, dtype=_i32)
    w = lax.axis_index("s") * NC + lax.axis_index("c")
    base = w * C

    neg_inf = jnp.full((L,), -jnp.inf, dtype=_f32)
    pos_inf = jnp.full((L,), jnp.inf, dtype=_f32)
    cur_cp = pltpu.async_copy(
        cur_hbm.at[pl.ds(pl.multiple_of(base, 8), C)], cur_buf, sem_cur)

    target = jnp.where(lane < 8, base, base + C).astype(_i32)
    k_vec = lane & 7
    group_a = lane < 8

    INIT_CHUNK = C // 7

    def ary_step(lo, hi, geometric, fill_step=None):
        w = hi - lo
        off = ((w * k_vec) >> 3) if geometric else k_vec
        p = lo + off
        pc = jnp.minimum(p, N_TICKS - 1)
        cp = pltpu.async_copy(ids_hbm.at[pc], probe_buf, sem_probe)
        if fill_step is not None:
            def init_body(j, carry):
                o = pl.multiple_of((fill_step * INIT_CHUNK + j) * L, L)
                acc_hi0[pl.ds(o, L)] = neg_inf
                acc_lo0[pl.ds(o, L)] = pos_inf
                return carry

            lax.fori_loop(0, INIT_CHUNK, init_body, 0)
        cp.wait()
        g = probe_buf[...]
        pred = (g < target) & (off < w)
        m_a = plsc.all_reduce_population_count(pred & group_a)[0]
        m_b = plsc.all_reduce_population_count(pred & ~group_a)[0]
        lo_a, hi_a, w_a = lo[0], hi[0], w[0]
        lo_b, hi_b, w_b = lo[8], hi[8], w[8]
        if geometric:
            nlo_a = jnp.where(m_a > 0, lo_a + ((w_a * (m_a - 1)) >> 3) + 1, lo_a)
            nhi_a = jnp.where(m_a < 8, lo_a + ((w_a * m_a) >> 3), hi_a)
            nlo_b = jnp.where(m_b > 0, lo_b + ((w_b * (m_b - 1)) >> 3) + 1, lo_b)
            nhi_b = jnp.where(m_b < 8, lo_b + ((w_b * m_b) >> 3), hi_b)
        else:
            nlo_a = lo_a + m_a
            nhi_a = nlo_a
            nlo_b = lo_b + m_b
            nhi_b = nlo_b
        nlo = jnp.where(group_a, nlo_a, nlo_b).astype(_i32)
        nhi = jnp.where(group_a, nhi_a, nhi_b).astype(_i32)
        return nlo, nhi

    lo0 = jnp.zeros((L,), dtype=_i32)
    hi0 = jnp.full((L,), N_TICKS, dtype=_i32)
    lo_v, hi_v = lax.fori_loop(
        0, 7, lambda s_, c: ary_step(c[0], c[1], True, fill_step=s_),
        (lo0, hi0))
    lo_v, hi_v = ary_step(lo_v, hi_v, False)
    t0 = lo_v[0]
    t1 = lo_v[8]

    start = (t0 // BLK) * BLK
    t1c = jnp.minimum(t1, ALIGNED)
    nblk = jnp.maximum((t1c - start + (BLK - 1)) // BLK, 0)

    UNROLL = 8

    bufs = ((ids_buf0, prc_buf0), (ids_buf1, prc_buf1))

    def issue(bi, slot, sem):
        off = jnp.minimum(start + bi * BLK, ALIGNED - BLK)
        off = pl.multiple_of(off, 8)
        pltpu.async_copy(ids_hbm.at[pl.ds(off, BLK)], bufs[slot][0], sem)
        pltpu.async_copy(prc_hbm.at[pl.ds(off, BLK)], bufs[slot][1], sem)

    def wait_blk(slot, sem):
        pltpu.make_async_copy(
            ids_hbm.at[pl.ds(0, BLK)], bufs[slot][0], sem).wait()
        pltpu.make_async_copy(
            prc_hbm.at[pl.ds(0, BLK)], bufs[slot][1], sem).wait()

    def flush(pid, prm, prn, cont):
        ploc = pid - base
        pvalid = (ploc >= 0) & (ploc < C)
        pslot = jnp.where(pvalid, ploc, 0) * L + lane
        fmask = jnp.logical_not(cont) & pvalid
        plsc.store_scatter(acc_hi0, [pslot], prm, mask=fmask)
        plsc.store_scatter(acc_lo0, [pslot], prn, mask=fmask)

    def step_vec(idv, pv, carry):
        pid, prm, prn = carry
        cont = idv == pid
        flush(pid, prm, prn, cont)
        rm = jnp.maximum(jnp.where(cont, prm, neg_inf), pv)
        rn = jnp.minimum(jnp.where(cont, prn, pos_inf), pv)
        return idv, rm, rn

    def process_block(slot, carry):
        def vec_body(i, c):
            for u in range(UNROLL):
                iu = i * UNROLL + u
                voff = pl.multiple_of(iu * L, L)
                idv = bufs[slot][0][pl.ds(voff, L)]
                pv = bufs[slot][1][pl.ds(voff, L)]
                c = step_vec(idv, pv, c)
            return c

        return lax.fori_loop(0, BLK // (L * UNROLL), vec_body, carry)

    def maybe_process(slot, cond, carry):
        return lax.cond(cond, lambda c: process_block(slot, c),
                        lambda c: c, carry)

    nb2 = (jnp.maximum(nblk, 1) + 1) >> 1
    issue(0, 0, sem_blk0)
    sentinel = jnp.full((L,), -1, dtype=_i32)
    carry0 = (sentinel, neg_inf, pos_inf)

    def pair_body(p2, carry):
        b0 = 2 * p2
        issue(b0 + 1, 1, sem_blk1)
        wait_blk(0, sem_blk0)
        carry = maybe_process(0, b0 < nblk, carry)

        @pl.when(p2 + 1 < nb2)
        def _prefetch():
            issue(b0 + 2, 0, sem_blk0)

        wait_blk(1, sem_blk1)
        carry = maybe_process(1, b0 + 1 < nblk, carry)
        return carry

    pid_f, prm_f, prn_f = lax.fori_loop(0, nb2, pair_body, carry0)
    flush(pid_f, prm_f, prn_f, jnp.zeros((L,), dtype=jnp.bool_))

    pltpu.async_copy(ids_hbm.at[pl.ds(ALIGNED, TAIL)],
                     ids_buf0.at[pl.ds(0, TAIL)], sem_blk0)
    pltpu.async_copy(prc_hbm.at[pl.ds(ALIGNED, TAIL)],
                     prc_buf0.at[pl.ds(0, TAIL)], sem_blk0)
    pltpu.make_async_copy(ids_hbm.at[pl.ds(0, TAIL)],
                          ids_buf0.at[pl.ds(0, TAIL)], sem_blk0).wait()
    pltpu.make_async_copy(prc_hbm.at[pl.ds(0, TAIL)],
                          prc_buf0.at[pl.ds(0, TAIL)], sem_blk0).wait()

    def tail_body(i, carry):
        for u in range(UNROLL):
            iu = i * UNROLL + u
            voff = pl.multiple_of(iu * L, L)
            idv = ids_buf0[pl.ds(voff, L)]
            pv = prc_buf0[pl.ds(voff, L)]
            loc = idv - base
            valid = (loc >= 0) & (loc < C)
            locc = jnp.where(valid, loc, 0)
            perm = (lane + iu) & (L - 1)
            slot_v = locc * L + perm
            h = plsc.load_gather(acc_hi0, [slot_v])
            lw = plsc.load_gather(acc_lo0, [slot_v])
            plsc.store_scatter(acc_hi0, [slot_v], jnp.maximum(h, pv), mask=valid)
            plsc.store_scatter(acc_lo0, [slot_v], jnp.minimum(lw, pv), mask=valid)
        return carry

    lax.fori_loop(0, TAIL // (L * UNROLL), tail_body, 0)

    one = jnp.float32(1.0)
    zero = jnp.float32(0.0)

    cur_cp.wait()

    def comb_body(j, carry):
        joff = j * L
        rowidx = (joff + lane) * L
        h = None
        lw = None
        for p in range(L):
            sk = rowidx + ((lane + p) & (L - 1))
            hp = plsc.load_gather(acc_hi0, [sk])
            lp = plsc.load_gather(acc_lo0, [sk])
            h = hp if h is None else jnp.maximum(h, hp)
            lw = lp if lw is None else jnp.minimum(lw, lp)
        empty = h == -jnp.inf
        h = jnp.where(empty, zero, h)
        lw = jnp.where(empty, zero, lw)
        cur = cur_buf[pl.ds(pl.multiple_of(joff, L), L)]
        sig = jnp.where(cur > h, one, jnp.where(cur < lw, -one, zero))
        sig_buf[pl.ds(pl.multiple_of(joff, L), L)] = sig
        return carry

    lax.fori_loop(0, C // L, comb_body, 0)
    pltpu.sync_copy(sig_buf, out_hbm.at[pl.ds(pl.multiple_of(base, 8), C)])


@jax.jit
def _run(inst_ids, tick_price, cur_price):
    mesh = plsc.VectorSubcoreMesh(core_axis_name="c", subcore_axis_name="s")
    kern = functools.partial(
        pl.kernel,
        mesh=mesh,
        compiler_params=pltpu.CompilerParams(needs_layout_passes=False),
        out_type=jax.ShapeDtypeStruct((NIDS,), _f32),
        scratch_types=[
            pltpu.VMEM((L * C,), _f32),
            pltpu.VMEM((L * C,), _f32),
            pltpu.VMEM((BLK,), _i32),
            pltpu.VMEM((BLK,), _i32),
            pltpu.VMEM((BLK,), _f32),
            pltpu.VMEM((BLK,), _f32),
            pltpu.VMEM((C,), _f32),
            pltpu.VMEM((C,), _f32),
            pltpu.VMEM((L,), _i32),
            pltpu.SemaphoreType.DMA,
            pltpu.SemaphoreType.DMA,
            pltpu.SemaphoreType.DMA,
            pltpu.SemaphoreType.DMA,
        ],
    )(_sc_body)
    cur_pad = jnp.concatenate(
        [cur_price, jnp.zeros((NIDS - NUM_INST,), dtype=_f32)])
    out = kern(inst_ids, tick_price, cur_pad)
    return out[:NUM_INST]


def kernel(timestamp, inst_ids, tick_price, cur_price):
    del timestamp
    return _run(inst_ids.astype(_i32), tick_price, cur_price)
